# baseline (speedup 1.0000x reference)
import functools

import jax
import jax.numpy as jnp
from jax import lax
from jax.experimental import pallas as pl
from jax.experimental.pallas import tpu as pltpu
from jax.experimental.pallas import tpu_sc as plsc

_info = plsc.get_sparse_core_info()
_NC = _info.num_cores
_NS = _info.num_subcores

_H = 200
_B = 4096
_D = 64
_V = 1_000_000

_BPT = _B // _NS
_HALF = 128
_NSUB = _H * _HALF
_DPC = _D // _NC
_VP = 1000064
_STC = 62464
_HQ = _H // 4
_NQTR = _HQ * _HALF


@functools.partial(
    pl.kernel,
    out_type=jax.ShapeDtypeStruct((_H, _D, _B), jnp.float32),
    mesh=plsc.VectorSubcoreMesh(core_axis_name="c", subcore_axis_name="s"),
    compiler_params=pltpu.CompilerParams(use_tc_tiling_on_sc=True),
    scratch_types=[
        pltpu.VMEM((2 * _NSUB,), jnp.int32),
        pltpu.VMEM((_NQTR,), jnp.float32),
        pltpu.VMEM((_NQTR,), jnp.float32),
        pltpu.SemaphoreType.DMA,
        pltpu.SemaphoreType.DMA,
        pltpu.SemaphoreType.DMA,
        pltpu.SemaphoreType.DMA,
        pltpu.VMEM_SHARED((_VP,), jnp.float32),
    ],
)
def _sc_lookup(tbl_t, idx_t, out_p, idx1, dstA, dstB,
               sem_st, sem_g, sem_oA, sem_oB, sp0):
    c = lax.axis_index("c")
    s = lax.axis_index("s")
    b0 = s * _BPT

    def load_blk(blk, carry):
        def fire(h, carry2):
            pltpu.async_copy(idx_t.at[h, pl.ds(b0, _HALF)],
                             idx1.at[pl.ds(h * _HALF, _HALF)], sem_st)
            pltpu.async_copy(idx_t.at[h, pl.ds(b0 + _HALF, _HALF)],
                             idx1.at[pl.ds(_NSUB + h * _HALF, _HALF)], sem_st)
            return carry2

        lax.fori_loop(blk * 25, blk * 25 + 25, fire, 0)

        def drain(h, carry2):
            pltpu.make_async_copy(idx_t.at[0, pl.ds(0, _HALF)],
                                  idx1.at[pl.ds(0, _HALF)], sem_st).wait()
            pltpu.make_async_copy(idx_t.at[0, pl.ds(0, _HALF)],
                                  idx1.at[pl.ds(0, _HALF)], sem_st).wait()
            return carry2

        lax.fori_loop(0, 25, drain, 0)
        return carry

    lax.fori_loop(0, _H // 25, load_blk, 0)

    def stage_start(dd):
        row = tbl_t.at[c * _DPC + dd]
        off = s * _STC
        pltpu.async_copy(row.at[pl.ds(off, _STC)],
                         sp0.at[pl.ds(off, _STC)], sem_st)

        @pl.when(s < 5)
        def _():
            toff = _NS * _STC + s * _HALF
            pltpu.async_copy(row.at[pl.ds(toff, _HALF)],
                             sp0.at[pl.ds(toff, _HALF)], sem_st)

    def stage_wait():
        pltpu.make_async_copy(tbl_t.at[0].at[pl.ds(0, _STC)],
                              sp0.at[pl.ds(0, _STC)], sem_st).wait()

        @pl.when(s < 5)
        def _():
            pltpu.make_async_copy(tbl_t.at[0].at[pl.ds(0, _HALF)],
                                  sp0.at[pl.ds(0, _HALF)], sem_st).wait()

    dsts = (dstA, dstB)
    osems = (sem_oA, sem_oB)

    def gather(half, q, buf):
        src = sp0.at[idx1.at[pl.ds(half * _NSUB + q * _NQTR, _NQTR)]]
        pltpu.async_copy(src, dsts[buf], sem_g)
        pltpu.make_async_copy(src, dsts[buf], sem_g).wait()

    def fire_writes(dd, half, q, buf):
        d = c * _DPC + dd
        bh = b0 + half * _HALF
        h0 = q * _HQ

        def wr(h, carry):
            pltpu.async_copy(dsts[buf].at[pl.ds(h * _HALF, _HALF)],
                             out_p.at[h0 + h, d, pl.ds(bh, _HALF)], osems[buf])
            return carry

        lax.fori_loop(0, _HQ, wr, 0)

    def drain_writes(buf):
        def dr(h, carry):
            pltpu.make_async_copy(dsts[buf].at[pl.ds(0, _HALF)],
                                  out_p.at[0, 0, pl.ds(0, _HALF)],
                                  osems[buf]).wait()
            return carry

        lax.fori_loop(0, _HQ, dr, 0)

    def body(dd, carry):
        stage_start(dd)
        stage_wait()
        plsc.subcore_barrier()

        for part in range(8):
            half, q = part // 4, part % 4
            buf = part % 2
            if part < 2:
                @pl.when(dd >= 1)
                def _():
                    drain_writes(buf)
            else:
                drain_writes(buf)

            gather(half, q, buf)
            fire_writes(dd, half, q, buf)

        plsc.subcore_barrier()
        return carry

    lax.fori_loop(0, _DPC, body, 0)
    drain_writes(0)
    drain_writes(1)


def kernel(indices, table):
    out_p = _sc_lookup(table.T, indices.T.astype(jnp.int32))
    return jnp.transpose(out_p, (2, 0, 1))


# speedup vs baseline: 1.0212x; 1.0212x over previous
import functools

import jax
import jax.numpy as jnp
from jax import lax
from jax.experimental import pallas as pl
from jax.experimental.pallas import tpu as pltpu
from jax.experimental.pallas import tpu_sc as plsc

_info = plsc.get_sparse_core_info()
_NC = _info.num_cores
_NS = _info.num_subcores

_H = 200
_B = 4096
_D = 64
_V = 1_000_000

_BPT = _B // _NS
_HALF = 128
_NSUB = _H * _HALF
_DPC = _D // _NC
_VP = 1000064
_STC = 62464
_HQ = _H // 2
_NQTR = _HQ * _HALF


@functools.partial(
    pl.kernel,
    out_type=jax.ShapeDtypeStruct((_H, _D, _B), jnp.float32),
    mesh=plsc.VectorSubcoreMesh(core_axis_name="c", subcore_axis_name="s"),
    compiler_params=pltpu.CompilerParams(use_tc_tiling_on_sc=True),
    scratch_types=[
        pltpu.VMEM((2 * _NSUB,), jnp.int32),
        pltpu.VMEM((_NQTR,), jnp.float32),
        pltpu.SemaphoreType.DMA,
        pltpu.SemaphoreType.DMA,
        pltpu.SemaphoreType.DMA,
        pltpu.VMEM_SHARED((_VP,), jnp.float32),
    ],
)
def _sc_lookup(tbl_t, idx_t, out_p, idx1, dstA,
               sem_st, sem_g, sem_oA, sp0):
    c = lax.axis_index("c")
    s = lax.axis_index("s")
    b0 = s * _BPT

    def load_blk(blk, carry):
        def fire(h, carry2):
            pltpu.async_copy(idx_t.at[h, pl.ds(b0, _HALF)],
                             idx1.at[pl.ds(h * _HALF, _HALF)], sem_st)
            pltpu.async_copy(idx_t.at[h, pl.ds(b0 + _HALF, _HALF)],
                             idx1.at[pl.ds(_NSUB + h * _HALF, _HALF)], sem_st)
            return carry2

        lax.fori_loop(blk * 25, blk * 25 + 25, fire, 0)

        def drain(h, carry2):
            pltpu.make_async_copy(idx_t.at[0, pl.ds(0, _HALF)],
                                  idx1.at[pl.ds(0, _HALF)], sem_st).wait()
            pltpu.make_async_copy(idx_t.at[0, pl.ds(0, _HALF)],
                                  idx1.at[pl.ds(0, _HALF)], sem_st).wait()
            return carry2

        lax.fori_loop(0, 25, drain, 0)
        return carry

    lax.fori_loop(0, _H // 25, load_blk, 0)

    def stage_start(dd):
        row = tbl_t.at[c * _DPC + dd]
        off = s * _STC
        pltpu.async_copy(row.at[pl.ds(off, _STC)],
                         sp0.at[pl.ds(off, _STC)], sem_st)

        @pl.when(s < 5)
        def _():
            toff = _NS * _STC + s * _HALF
            pltpu.async_copy(row.at[pl.ds(toff, _HALF)],
                             sp0.at[pl.ds(toff, _HALF)], sem_st)

    def stage_wait():
        pltpu.make_async_copy(tbl_t.at[0].at[pl.ds(0, _STC)],
                              sp0.at[pl.ds(0, _STC)], sem_st).wait()

        @pl.when(s < 5)
        def _():
            pltpu.make_async_copy(tbl_t.at[0].at[pl.ds(0, _HALF)],
                                  sp0.at[pl.ds(0, _HALF)], sem_st).wait()

    dsts = (dstA,)
    osems = (sem_oA,)

    def gather(half, q, buf):
        src = sp0.at[idx1.at[pl.ds(half * _NSUB + q * _NQTR, _NQTR)]]
        pltpu.async_copy(src, dsts[buf], sem_g)
        pltpu.make_async_copy(src, dsts[buf], sem_g).wait()

    def fire_writes(dd, half, q, buf):
        d = c * _DPC + dd
        bh = b0 + half * _HALF
        h0 = q * _HQ

        def wr(h, carry):
            pltpu.async_copy(dsts[buf].at[pl.ds(h * _HALF, _HALF)],
                             out_p.at[h0 + h, d, pl.ds(bh, _HALF)], osems[buf])
            return carry

        lax.fori_loop(0, _HQ, wr, 0)

    def drain_writes(buf):
        def dr(h, carry):
            pltpu.make_async_copy(dsts[buf].at[pl.ds(0, _HALF)],
                                  out_p.at[0, 0, pl.ds(0, _HALF)],
                                  osems[buf]).wait()
            return carry

        lax.fori_loop(0, _HQ, dr, 0)

    def body(dd, carry):
        stage_start(dd)
        stage_wait()
        plsc.subcore_barrier()

        for part in range(4):
            half, q = part // 2, part % 2
            buf = 0
            if part == 0:
                @pl.when(dd >= 1)
                def _():
                    drain_writes(buf)
            else:
                drain_writes(buf)

            gather(half, q, buf)
            fire_writes(dd, half, q, buf)

        plsc.subcore_barrier()
        return carry

    lax.fori_loop(0, _DPC, body, 0)
    drain_writes(0)


def kernel(indices, table):
    out_p = _sc_lookup(table.T, indices.T.astype(jnp.int32))
    return jnp.transpose(out_p, (2, 0, 1))
